# Initial kernel scaffold; baseline (speedup 1.0000x reference)
import jax
import jax.numpy as jnp
from jax.experimental import pallas as pl


def kernel(x, W1, b1, W2, b2, W3, b3, D1, c1, D2, c2, D3, c3, emb_w):
    raise NotImplementedError("write your pallas kernel here")


# speedup vs baseline: 1.3684x; 1.3684x over previous
import functools

import jax
import jax.numpy as jnp
from jax.experimental import pallas as pl

B_TOTAL = 4096
BB = 256
K = 8192
EMB = 32


def _lrelu(v):
    return jnp.where(v > 0, v, 0.01 * v)


def _dot(a, b):
    return jnp.dot(a, b, preferred_element_type=jnp.float32)


def _vq_kernel(x_ref, w1_ref, b1_ref, w2_ref, b2_ref, w3a_ref, b3a_ref,
               w3b_ref, b3b_ref, d1a_ref, d1b_ref, c1_ref, d2_ref, c2_ref,
               d3_ref, c3_ref, embw_ref, embwt_ref,
               i0_ref, i1_ref, z0_ref, z1_ref, q0_ref, q1_ref,
               zq0_ref, zq1_ref, xp_ref):
    x = x_ref[...]
    h1 = _lrelu(_dot(x, w1_ref[...]) + b1_ref[...])
    h2 = _lrelu(_dot(h1, w2_ref[...]) + b2_ref[...])
    z0 = _lrelu(_dot(h2, w3a_ref[...]) + b3a_ref[...])
    z1 = _lrelu(_dot(h2, w3b_ref[...]) + b3b_ref[...])
    z0_ref[...] = z0
    z1_ref[...] = z1

    embw = embw_ref[...]
    w2sum = jnp.sum(embw * embw, axis=0)[None, :]
    embwt = embwt_ref[...]

    def nearest(z):
        x2 = jnp.sum(z * z, axis=1, keepdims=True)
        dist = (x2 - 2.0 * _dot(z, embw)) + w2sum
        m = jnp.min(dist, axis=1, keepdims=True)
        lane = jax.lax.broadcasted_iota(jnp.int32, dist.shape, 1)
        idx = jnp.min(jnp.where(dist <= m, lane, K), axis=1)
        onehot = (lane == idx[:, None]).astype(jnp.float32)
        q = _dot(onehot, embwt)
        return idx, q

    idx0, q0 = nearest(z0)
    idx1, q1 = nearest(z1)
    i0_ref[0, 0, :] = idx0
    i1_ref[0, 0, :] = idx1
    q0_ref[...] = q0
    q1_ref[...] = q1
    zq0 = z0 + (q0 - z0)
    zq1 = z1 + (q1 - z1)
    zq0_ref[...] = zq0
    zq1_ref[...] = zq1

    g1 = _lrelu(_dot(zq0, d1a_ref[...]) + _dot(zq1, d1b_ref[...])
                + c1_ref[...])
    g2 = _lrelu(_dot(g1, d2_ref[...]) + c2_ref[...])
    xp_ref[...] = jax.nn.sigmoid(_dot(g2, d3_ref[...]) + c3_ref[...])


@jax.jit
def kernel(x, W1, b1, W2, b2, W3, b3, D1, c1, D2, c2, D3, c3, emb_w):
    B = x.shape[0]
    nb = B // BB
    W3a, W3b = W3[:, 0::2], W3[:, 1::2]
    b3a, b3b = b3[0::2][None, :], b3[1::2][None, :]
    D1a, D1b = D1[0::2, :], D1[1::2, :]
    emb_wT = emb_w.T

    row_spec = lambda w: pl.BlockSpec((BB, w), lambda i: (i, 0))
    full = lambda a: pl.BlockSpec(a.shape, lambda i: (0,) * a.ndim)
    idx_spec = pl.BlockSpec((1, 1, BB), lambda i: (i, 0, 0))
    f32 = jnp.float32

    outs = pl.pallas_call(
        _vq_kernel,
        grid=(nb,),
        in_specs=[
            row_spec(x.shape[1]),
            full(W1), full(b1[None, :]), full(W2), full(b2[None, :]),
            full(W3a), full(b3a), full(W3b), full(b3b),
            full(D1a), full(D1b), full(c1[None, :]),
            full(D2), full(c2[None, :]), full(D3), full(c3[None, :]),
            full(emb_w), full(emb_wT),
        ],
        out_specs=[
            idx_spec, idx_spec,
            row_spec(EMB), row_spec(EMB), row_spec(EMB), row_spec(EMB),
            row_spec(EMB), row_spec(EMB), row_spec(x.shape[1]),
        ],
        out_shape=[
            jax.ShapeDtypeStruct((nb, 1, BB), jnp.int32),
            jax.ShapeDtypeStruct((nb, 1, BB), jnp.int32),
            jax.ShapeDtypeStruct((B, EMB), f32),
            jax.ShapeDtypeStruct((B, EMB), f32),
            jax.ShapeDtypeStruct((B, EMB), f32),
            jax.ShapeDtypeStruct((B, EMB), f32),
            jax.ShapeDtypeStruct((B, EMB), f32),
            jax.ShapeDtypeStruct((B, EMB), f32),
            jax.ShapeDtypeStruct((B, x.shape[1]), f32),
        ],
    )(x, W1, b1[None, :], W2, b2[None, :], W3a, b3a, W3b, b3b,
      D1a, D1b, c1[None, :], D2, c2[None, :], D3, c3[None, :],
      emb_w, emb_wT)

    i0, i1, z0, z1, q0, q1, zq0, zq1, xp = outs
    idx = jnp.stack([i0.reshape(B), i1.reshape(B)], axis=1)
    z_e = jnp.stack([z0, z1], axis=-1)
    z_q = jnp.stack([zq0, zq1], axis=-1)
    emb = jnp.stack([q0, q1], axis=-1)
    return idx, z_e, z_q, emb, xp
